# Initial kernel scaffold; baseline (speedup 1.0000x reference)
import jax
import jax.numpy as jnp
from jax.experimental import pallas as pl


def kernel(x, edge_attr, edge_index, msg_w0, msg_b0, agg_w0, agg_b0, msg_w1, msg_b1, agg_w1, agg_b1, msg_w2, msg_b2, agg_w2, agg_b2, eu_w0, eu_b0, eu_w1, eu_b1, eu_w2, eu_b2, post_w0, post_b0, post_w1, post_b1):
    raise NotImplementedError("write your pallas kernel here")


# speedup vs baseline: 2.2036x; 2.2036x over previous
import functools

import jax
import jax.numpy as jnp
from jax import lax
from jax.experimental import pallas as pl
from jax.experimental.pallas import tpu as pltpu
from jax.experimental.pallas import tpu_sc as plsc

NC = 2
NS = 16
NW = NC * NS
CH = 128
D = 64


RC = 16


def _row_sweep(n, s, do):
    nrch = n // RC
    rr = nrch // NS
    rrem = nrch - rr * NS

    @pl.loop(0, rr)
    def _(t):
        do((t * NS + s) * RC)

    if rrem:
        @pl.when(s < rrem)
        def _():
            do((rr * NS + s) * RC)


@functools.lru_cache(maxsize=None)
def _conv_sc(n, e, with_cnt):
    assert e % CH == 0 and n % RC == 0
    nchunks = e // CH
    rounds = nchunks // NW
    rem = nchunks - rounds * NW

    mesh = plsc.VectorSubcoreMesh(core_axis_name="c", subcore_axis_name="s",
                                  num_cores=NC, num_subcores=NS)
    out_type = [jax.ShapeDtypeStruct((NC, n, D), jnp.float32)]
    scratch = [
        pltpu.VMEM_SHARED((n, D), jnp.float32),
        pltpu.VMEM_SHARED((n, D), jnp.float32),
        pltpu.VMEM((CH,), jnp.int32),
        pltpu.VMEM((CH,), jnp.int32),
        pltpu.VMEM((CH, D), jnp.float32),
        pltpu.VMEM((CH, D), jnp.float32),
        pltpu.SemaphoreType.DMA,
    ]
    if with_cnt:
        out_type.append(jax.ShapeDtypeStruct((NC, n, 16), jnp.float32))
        scratch += [
            pltpu.VMEM_SHARED((n, 16), jnp.float32),
            pltpu.VMEM((CH, 16), jnp.float32),
        ]

    def body(src_h, dst_h, tab_h, eam_h, z64_h, z16_h, *rest):
        if with_cnt:
            out_h, cnt_h, tab_s, acc_s, sidx, didx, g, ebuf, sem, cnt_s, ones = rest
        else:
            out_h, tab_s, acc_s, sidx, didx, g, ebuf, sem = rest
        c = lax.axis_index("c")
        s = lax.axis_index("s")
        wid = c * NS + s

        def stage(rb):
            sl = pl.ds(rb, RC)
            pltpu.sync_copy(tab_h.at[sl], tab_s.at[sl])
            pltpu.sync_copy(z64_h.at[sl], acc_s.at[sl])
            if with_cnt:
                pltpu.sync_copy(z16_h.at[sl], cnt_s.at[sl])

        _row_sweep(n, s, stage)
        if with_cnt:
            @pl.loop(0, CH)
            def _(i):
                ones[i, :] = jnp.full((16,), 1.0, jnp.float32)

        plsc.subcore_barrier()

        def do_chunk(base):
            pltpu.sync_copy(src_h.at[pl.ds(base, CH)], sidx)
            pltpu.sync_copy(dst_h.at[pl.ds(base, CH)], didx)
            cp = pltpu.async_copy(tab_s.at[sidx], g, sem)
            pltpu.sync_copy(eam_h.at[pl.ds(base, CH)], ebuf)
            cp.wait()

            @pl.loop(0, CH)
            def _(i):
                for j in range(D // 16):
                    sl = pl.ds(j * 16, 16)
                    g[i, sl] = jnp.maximum(g[i, sl] + ebuf[i, sl], 0.0)

            pltpu.sync_copy(g, acc_s.at[didx], add=True)
            if with_cnt:
                pltpu.sync_copy(ones, cnt_s.at[didx], add=True)

        @pl.loop(0, rounds)
        def _(t):
            do_chunk((t * NW + wid) * CH)

        if rem:
            @pl.when(wid < rem)
            def _():
                do_chunk((rounds * NW + wid) * CH)

        plsc.subcore_barrier()

        def writeback(rb):
            sl = pl.ds(rb, RC)
            pltpu.sync_copy(acc_s.at[sl], out_h.at[c, sl])
            if with_cnt:
                pltpu.sync_copy(cnt_s.at[sl], cnt_h.at[c, sl])

        _row_sweep(n, s, writeback)

    return pl.kernel(body, out_type=out_type, mesh=mesh, scratch_types=scratch,
                     compiler_params=pltpu.CompilerParams(use_tc_tiling_on_sc=False))


@functools.lru_cache(maxsize=None)
def _eupd_sc(n, e):
    assert e % CH == 0 and n % RC == 0
    nchunks = e // CH
    rounds = nchunks // NW
    rem = nchunks - rounds * NW

    mesh = plsc.VectorSubcoreMesh(core_axis_name="c", subcore_axis_name="s",
                                  num_cores=NC, num_subcores=NS)
    out_type = [jax.ShapeDtypeStruct((e, 16), jnp.float32)]
    scratch = [
        pltpu.VMEM_SHARED((n, 16), jnp.float32),
        pltpu.VMEM_SHARED((n, 16), jnp.float32),
        pltpu.VMEM((CH,), jnp.int32),
        pltpu.VMEM((CH,), jnp.int32),
        pltpu.VMEM((CH, 16), jnp.float32),
        pltpu.VMEM((CH, 16), jnp.float32),
        pltpu.SemaphoreType.DMA,
        pltpu.SemaphoreType.DMA,
    ]

    def body(src_h, dst_h, ti_h, tj_h, out_h, ti_s, tj_s, sidx, didx, gi, gj,
             sem0, sem1):
        c = lax.axis_index("c")
        s = lax.axis_index("s")
        wid = c * NS + s

        def stage(rb):
            sl = pl.ds(rb, RC)
            pltpu.sync_copy(ti_h.at[sl], ti_s.at[sl])
            pltpu.sync_copy(tj_h.at[sl], tj_s.at[sl])

        _row_sweep(n, s, stage)
        plsc.subcore_barrier()

        def do_chunk(base):
            pltpu.sync_copy(src_h.at[pl.ds(base, CH)], sidx)
            pltpu.sync_copy(dst_h.at[pl.ds(base, CH)], didx)
            cp0 = pltpu.async_copy(ti_s.at[sidx], gi, sem0)
            cp1 = pltpu.async_copy(tj_s.at[didx], gj, sem1)
            cp0.wait()
            cp1.wait()

            @pl.loop(0, CH)
            def _(i):
                gi[i, :] = gi[i, :] + gj[i, :]

            pltpu.sync_copy(gi, out_h.at[pl.ds(base, CH)])

        @pl.loop(0, rounds)
        def _(t):
            do_chunk((t * NW + wid) * CH)

        if rem:
            @pl.when(wid < rem)
            def _():
                do_chunk((rounds * NW + wid) * CH)

    return pl.kernel(body, out_type=out_type, mesh=mesh, scratch_types=scratch,
                     compiler_params=pltpu.CompilerParams(use_tc_tiling_on_sc=False))


BE = 6400


def _edge0_body(ea_ref, wm_ref, bm_ref, we_ref, be_ref, eam_ref, eae_ref):
    a = ea_ref[...]
    eam_ref[...] = jnp.dot(a, wm_ref[...], preferred_element_type=jnp.float32) + bm_ref[...]
    eae_ref[...] = jnp.dot(a, we_ref[...], preferred_element_type=jnp.float32) + be_ref[...]


@functools.lru_cache(maxsize=None)
def _edge0_call(e):
    assert e % BE == 0
    return pl.pallas_call(
        _edge0_body,
        grid=(e // BE,),
        in_specs=[
            pl.BlockSpec((BE, 16), lambda i: (i, 0)),
            pl.BlockSpec((16, D), lambda i: (0, 0)),
            pl.BlockSpec((1, D), lambda i: (0, 0)),
            pl.BlockSpec((16, 16), lambda i: (0, 0)),
            pl.BlockSpec((1, 16), lambda i: (0, 0)),
        ],
        out_specs=[
            pl.BlockSpec((BE, D), lambda i: (i, 0)),
            pl.BlockSpec((BE, 16), lambda i: (i, 0)),
        ],
        out_shape=[
            jax.ShapeDtypeStruct((e, D), jnp.float32),
            jax.ShapeDtypeStruct((e, 16), jnp.float32),
        ],
    )


def _edge_mid_body2(pre_ref, eae_ref, wm_ref, bm_ref, we_ref, be_ref,
                    eam_ref, eaen_ref):
    a = jnp.maximum(pre_ref[...] + eae_ref[...], 0.0)
    eam_ref[...] = jnp.dot(a, wm_ref[...], preferred_element_type=jnp.float32) + bm_ref[...]
    eaen_ref[...] = jnp.dot(a, we_ref[...], preferred_element_type=jnp.float32) + be_ref[...]


def _edge_mid_body1(pre_ref, eae_ref, wm_ref, bm_ref, eam_ref):
    a = jnp.maximum(pre_ref[...] + eae_ref[...], 0.0)
    eam_ref[...] = jnp.dot(a, wm_ref[...], preferred_element_type=jnp.float32) + bm_ref[...]


@functools.lru_cache(maxsize=None)
def _edge_mid_call(e, with_next):
    assert e % BE == 0
    in_specs = [
        pl.BlockSpec((BE, 16), lambda i: (i, 0)),
        pl.BlockSpec((BE, 16), lambda i: (i, 0)),
        pl.BlockSpec((16, D), lambda i: (0, 0)),
        pl.BlockSpec((1, D), lambda i: (0, 0)),
    ]
    out_specs = [pl.BlockSpec((BE, D), lambda i: (i, 0))]
    out_shape = [jax.ShapeDtypeStruct((e, D), jnp.float32)]
    if with_next:
        in_specs += [
            pl.BlockSpec((16, 16), lambda i: (0, 0)),
            pl.BlockSpec((1, 16), lambda i: (0, 0)),
        ]
        out_specs.append(pl.BlockSpec((BE, 16), lambda i: (i, 0)))
        out_shape.append(jax.ShapeDtypeStruct((e, 16), jnp.float32))
    return pl.pallas_call(
        _edge_mid_body2 if with_next else _edge_mid_body1,
        grid=(e // BE,),
        in_specs=in_specs,
        out_specs=out_specs,
        out_shape=out_shape,
    )


def _proj_body(x_ref, w_ref, o_ref):
    o_ref[...] = jnp.dot(x_ref[...], w_ref[...], preferred_element_type=jnp.float32)


@functools.lru_cache(maxsize=None)
def _proj_call(n, din):
    return pl.pallas_call(
        _proj_body,
        out_shape=jax.ShapeDtypeStruct((n, D), jnp.float32),
    )


def _node_mid_body(s_ref, cnt_ref, x_ref, awm_ref, awx_ref, ab_ref,
                   wi_ref, wj_ref, mwx_ref, xo_ref, ti_ref, tj_ref, xm_ref):
    ssum = s_ref[0] + s_ref[1]
    cnt = cnt_ref[0, :, 0:1] + cnt_ref[1, :, 0:1]
    mean = ssum / jnp.maximum(cnt, 1.0)
    h = jnp.dot(mean, awm_ref[...], preferred_element_type=jnp.float32)
    h = h + jnp.dot(x_ref[...], awx_ref[...], preferred_element_type=jnp.float32)
    h = jnp.maximum(h + ab_ref[...], 0.0)
    nrm = jnp.sqrt(jnp.sum(h * h, axis=1, keepdims=True))
    xn = h / jnp.maximum(nrm, 1e-12)
    xo_ref[...] = xn
    ti_ref[...] = jnp.dot(xn, wi_ref[...], preferred_element_type=jnp.float32)
    tj_ref[...] = jnp.dot(xn, wj_ref[...], preferred_element_type=jnp.float32)
    xm_ref[...] = jnp.dot(xn, mwx_ref[...], preferred_element_type=jnp.float32)


@functools.lru_cache(maxsize=None)
def _node_mid_call(n, din):
    return pl.pallas_call(
        _node_mid_body,
        out_shape=[
            jax.ShapeDtypeStruct((n, D), jnp.float32),
            jax.ShapeDtypeStruct((n, 16), jnp.float32),
            jax.ShapeDtypeStruct((n, 16), jnp.float32),
            jax.ShapeDtypeStruct((n, D), jnp.float32),
        ],
    )


def _node_final_body(s_ref, cnt_ref, x_ref, awm_ref, awx_ref, ab_ref,
                     pw0_ref, pb0_ref, pw1_ref, pb1_ref, out_ref):
    ssum = s_ref[0] + s_ref[1]
    cnt = cnt_ref[0, :, 0:1] + cnt_ref[1, :, 0:1]
    mean = ssum / jnp.maximum(cnt, 1.0)
    h = jnp.dot(mean, awm_ref[...], preferred_element_type=jnp.float32)
    h = h + jnp.dot(x_ref[...], awx_ref[...], preferred_element_type=jnp.float32)
    h = jnp.maximum(h + ab_ref[...], 0.0)
    nrm = jnp.sqrt(jnp.sum(h * h, axis=1, keepdims=True))
    xn = h / jnp.maximum(nrm, 1e-12)
    o = jnp.maximum(jnp.dot(xn, pw0_ref[...], preferred_element_type=jnp.float32) + pb0_ref[...], 0.0)
    out_ref[...] = jnp.dot(o, pw1_ref[...], preferred_element_type=jnp.float32) + pb1_ref[...]


@functools.lru_cache(maxsize=None)
def _node_final_call(n):
    return pl.pallas_call(
        _node_final_body,
        out_shape=jax.ShapeDtypeStruct((n, D), jnp.float32),
    )


def kernel(x, edge_attr, edge_index,
           msg_w0, msg_b0, agg_w0, agg_b0,
           msg_w1, msg_b1, agg_w1, agg_b1,
           msg_w2, msg_b2, agg_w2, agg_b2,
           eu_w0, eu_b0, eu_w1, eu_b1, eu_w2, eu_b2,
           post_w0, post_b0, post_w1, post_b1):
    n, din = x.shape
    e = edge_attr.shape[0]
    src = edge_index[0].astype(jnp.int32)
    dst = edge_index[1].astype(jnp.int32)
    z64 = jnp.zeros((n, D), jnp.float32)
    z16 = jnp.zeros((n, 16), jnp.float32)

    r2 = lambda b: b.reshape(1, -1)

    eam0, eae0 = _edge0_call(e)(edge_attr, msg_w0[din:], r2(msg_b0),
                                eu_w0[2 * D:], r2(eu_b0))
    xm0 = _proj_call(n, din)(x, msg_w0[:din])

    part0, cntp = _conv_sc(n, e, True)(src, dst, xm0, eam0, z64, z16)
    x1, t1i, t1j, xm1 = _node_mid_call(n, din)(
        part0, cntp, x, agg_w0[:D], agg_w0[D:], r2(agg_b0),
        eu_w0[:D], eu_w0[D:2 * D], msg_w1[:D])

    (pre1,) = _eupd_sc(n, e)(src, dst, t1i, t1j)
    eam1, eae1 = _edge_mid_call(e, True)(pre1, eae0, msg_w1[D:], r2(msg_b1),
                                         eu_w1[2 * D:], r2(eu_b1))

    (part1,) = _conv_sc(n, e, False)(src, dst, xm1, eam1, z64, z16)
    x2, t2i, t2j, xm2 = _node_mid_call(n, D)(
        part1, cntp, x1, agg_w1[:D], agg_w1[D:], r2(agg_b1),
        eu_w1[:D], eu_w1[D:2 * D], msg_w2[:D])

    (pre2,) = _eupd_sc(n, e)(src, dst, t2i, t2j)
    (eam2,) = _edge_mid_call(e, False)(pre2, eae1, msg_w2[D:], r2(msg_b2))

    (part2,) = _conv_sc(n, e, False)(src, dst, xm2, eam2, z64, z16)
    out = _node_final_call(n)(
        part2, cntp, x2, agg_w2[:D], agg_w2[D:], r2(agg_b2),
        post_w0, r2(post_b0), post_w1, r2(post_b1))
    return out


# speedup vs baseline: 2.6698x; 1.2115x over previous
import functools

import jax
import jax.numpy as jnp
from jax import lax
from jax.experimental import pallas as pl
from jax.experimental.pallas import tpu as pltpu
from jax.experimental.pallas import tpu_sc as plsc

NC = 2
NS = 16
NW = NC * NS
CH = 128
D = 64


RC = 16


def _row_sweep(n, s, do):
    nrch = n // RC
    rr = nrch // NS
    rrem = nrch - rr * NS

    @pl.loop(0, rr)
    def _(t):
        do((t * NS + s) * RC)

    if rrem:
        @pl.when(s < rrem)
        def _():
            do((rr * NS + s) * RC)


@functools.lru_cache(maxsize=None)
def _conv_sc(n, e, with_cnt):
    assert e % CH == 0 and n % RC == 0
    nchunks = e // CH
    rounds = nchunks // NW
    rem = nchunks - rounds * NW

    mesh = plsc.VectorSubcoreMesh(core_axis_name="c", subcore_axis_name="s",
                                  num_cores=NC, num_subcores=NS)
    out_type = [jax.ShapeDtypeStruct((NC, n, D), jnp.float32)]
    scratch = [
        pltpu.VMEM_SHARED((n, D), jnp.float32),
        pltpu.VMEM_SHARED((n, D), jnp.float32),
        pltpu.VMEM((CH,), jnp.int32),
        pltpu.VMEM((CH,), jnp.int32),
        pltpu.VMEM((CH, D), jnp.float32),
        pltpu.VMEM((4, CH // 8, 128), jnp.float32),
        pltpu.SemaphoreType.DMA,
    ]
    if with_cnt:
        out_type.append(jax.ShapeDtypeStruct((NC, n, 16), jnp.float32))
        scratch += [
            pltpu.VMEM_SHARED((n, 16), jnp.float32),
            pltpu.VMEM((CH, 16), jnp.float32),
        ]

    def body(src_h, dst_h, tab_h, eam_h, z64_h, z16_h, *rest):
        if with_cnt:
            out_h, cnt_h, tab_s, acc_s, sidx, didx, g, ebuf, sem, cnt_s, ones = rest
        else:
            out_h, tab_s, acc_s, sidx, didx, g, ebuf, sem = rest
        c = lax.axis_index("c")
        s = lax.axis_index("s")
        wid = c * NS + s

        def stage(rb):
            sl = pl.ds(rb, RC)
            pltpu.sync_copy(tab_h.at[sl], tab_s.at[sl])
            pltpu.sync_copy(z64_h.at[sl], acc_s.at[sl])
            if with_cnt:
                pltpu.sync_copy(z16_h.at[sl], cnt_s.at[sl])

        _row_sweep(n, s, stage)
        if with_cnt:
            @pl.loop(0, CH)
            def _(i):
                ones[i, :] = jnp.full((16,), 1.0, jnp.float32)

        plsc.subcore_barrier()

        def do_chunk(base):
            pltpu.sync_copy(src_h.at[pl.ds(base, CH)], sidx)
            pltpu.sync_copy(dst_h.at[pl.ds(base, CH)], didx)
            cp = pltpu.async_copy(tab_s.at[sidx], g, sem)
            pltpu.sync_copy(eam_h.at[:, pl.ds(base // 8, CH // 8), :], ebuf)
            cp.wait()

            @pl.loop(0, CH // 8)
            def _(o):
                for u in range(8):
                    k = 8 * o + u
                    for j in range(D // 16):
                        g[k, pl.ds(j * 16, 16)] = jnp.maximum(
                            g[k, pl.ds(j * 16, 16)]
                            + ebuf[j, o, pl.ds(u * 16, 16)], 0.0)

            pltpu.sync_copy(g, acc_s.at[didx], add=True)
            if with_cnt:
                pltpu.sync_copy(ones, cnt_s.at[didx], add=True)

        @pl.loop(0, rounds)
        def _(t):
            do_chunk((t * NW + wid) * CH)

        if rem:
            @pl.when(wid < rem)
            def _():
                do_chunk((rounds * NW + wid) * CH)

        plsc.subcore_barrier()

        def writeback(rb):
            sl = pl.ds(rb, RC)
            pltpu.sync_copy(acc_s.at[sl], out_h.at[c, sl])
            if with_cnt:
                pltpu.sync_copy(cnt_s.at[sl], cnt_h.at[c, sl])

        _row_sweep(n, s, writeback)

    return pl.kernel(body, out_type=out_type, mesh=mesh, scratch_types=scratch,
                     compiler_params=pltpu.CompilerParams(use_tc_tiling_on_sc=False))


@functools.lru_cache(maxsize=None)
def _eupd_sc(n, e):
    assert e % CH == 0 and n % RC == 0
    nchunks = e // CH
    rounds = nchunks // NW
    rem = nchunks - rounds * NW

    mesh = plsc.VectorSubcoreMesh(core_axis_name="c", subcore_axis_name="s",
                                  num_cores=NC, num_subcores=NS)
    out_type = [jax.ShapeDtypeStruct((e // 8, 128), jnp.float32)]
    scratch = [
        pltpu.VMEM_SHARED((n, 16), jnp.float32),
        pltpu.VMEM_SHARED((n, 16), jnp.float32),
        pltpu.VMEM((CH,), jnp.int32),
        pltpu.VMEM((CH,), jnp.int32),
        pltpu.VMEM((CH, 16), jnp.float32),
        pltpu.VMEM((CH, 16), jnp.float32),
        pltpu.VMEM((CH // 8, 128), jnp.float32),
        pltpu.SemaphoreType.DMA,
        pltpu.SemaphoreType.DMA,
    ]

    def body(src_h, dst_h, ti_h, tj_h, out_h, ti_s, tj_s, sidx, didx, gi, gj,
             ob, sem0, sem1):
        c = lax.axis_index("c")
        s = lax.axis_index("s")
        wid = c * NS + s

        def stage(rb):
            sl = pl.ds(rb, RC)
            pltpu.sync_copy(ti_h.at[sl], ti_s.at[sl])
            pltpu.sync_copy(tj_h.at[sl], tj_s.at[sl])

        _row_sweep(n, s, stage)
        plsc.subcore_barrier()

        def do_chunk(base):
            pltpu.sync_copy(src_h.at[pl.ds(base, CH)], sidx)
            pltpu.sync_copy(dst_h.at[pl.ds(base, CH)], didx)
            cp0 = pltpu.async_copy(ti_s.at[sidx], gi, sem0)
            cp1 = pltpu.async_copy(tj_s.at[didx], gj, sem1)
            cp0.wait()
            cp1.wait()

            @pl.loop(0, CH // 8)
            def _(o):
                for u in range(8):
                    k = 8 * o + u
                    ob[o, pl.ds(u * 16, 16)] = gi[k, :] + gj[k, :]

            pltpu.sync_copy(ob, out_h.at[pl.ds(base // 8, CH // 8)])

        @pl.loop(0, rounds)
        def _(t):
            do_chunk((t * NW + wid) * CH)

        if rem:
            @pl.when(wid < rem)
            def _():
                do_chunk((rounds * NW + wid) * CH)

    return pl.kernel(body, out_type=out_type, mesh=mesh, scratch_types=scratch,
                     compiler_params=pltpu.CompilerParams(use_tc_tiling_on_sc=False))


BE = 6400


BP = BE // 8


def _edge0_body(ea_ref, wm8_ref, bm8_ref, we8_ref, be8_ref, eam_ref, eae_ref):
    a8 = ea_ref[...]
    for j in range(4):
        eam_ref[j] = jnp.dot(a8, wm8_ref[j], preferred_element_type=jnp.float32) + bm8_ref[j]
    eae_ref[...] = jnp.dot(a8, we8_ref[...], preferred_element_type=jnp.float32) + be8_ref[...]


def _edge_mid_body2(pre_ref, eae_ref, wm8_ref, bm8_ref, we8_ref, be8_ref,
                    eam_ref, eaen_ref):
    a8 = jnp.maximum(pre_ref[...] + eae_ref[...], 0.0)
    for j in range(4):
        eam_ref[j] = jnp.dot(a8, wm8_ref[j], preferred_element_type=jnp.float32) + bm8_ref[j]
    eaen_ref[...] = jnp.dot(a8, we8_ref[...], preferred_element_type=jnp.float32) + be8_ref[...]


def _edge_mid_body1(pre_ref, eae_ref, wm8_ref, bm8_ref, eam_ref):
    a8 = jnp.maximum(pre_ref[...] + eae_ref[...], 0.0)
    for j in range(4):
        eam_ref[j] = jnp.dot(a8, wm8_ref[j], preferred_element_type=jnp.float32) + bm8_ref[j]


_P8_IN = lambda: pl.BlockSpec((BP, 128), lambda i: (i, 0))
_WM8 = lambda: pl.BlockSpec((4, 128, 128), lambda i: (0, 0, 0))
_BM8 = lambda: pl.BlockSpec((4, 1, 128), lambda i: (0, 0, 0))
_WE8 = lambda: pl.BlockSpec((128, 128), lambda i: (0, 0))
_BE8 = lambda: pl.BlockSpec((1, 128), lambda i: (0, 0))


def _eam_out(e):
    return (pl.BlockSpec((4, BP, 128), lambda i: (0, i, 0)),
            jax.ShapeDtypeStruct((4, e // 8, 128), jnp.float32))


@functools.lru_cache(maxsize=None)
def _edge0_call(e):
    assert e % BE == 0
    eam_spec, eam_shape = _eam_out(e)
    return pl.pallas_call(
        _edge0_body,
        grid=(e // BE,),
        in_specs=[_P8_IN(), _WM8(), _BM8(), _WE8(), _BE8()],
        out_specs=[eam_spec, _P8_IN()],
        out_shape=[eam_shape, jax.ShapeDtypeStruct((e // 8, 128), jnp.float32)],
    )


@functools.lru_cache(maxsize=None)
def _edge_mid_call(e, with_next):
    assert e % BE == 0
    eam_spec, eam_shape = _eam_out(e)
    in_specs = [_P8_IN(), _P8_IN(), _WM8(), _BM8()]
    out_specs = [eam_spec]
    out_shape = [eam_shape]
    if with_next:
        in_specs += [_WE8(), _BE8()]
        out_specs.append(_P8_IN())
        out_shape.append(jax.ShapeDtypeStruct((e // 8, 128), jnp.float32))
    return pl.pallas_call(
        _edge_mid_body2 if with_next else _edge_mid_body1,
        grid=(e // BE,),
        in_specs=in_specs,
        out_specs=out_specs,
        out_shape=out_shape,
    )


def _proj_body(x_ref, w_ref, o_ref):
    o_ref[...] = jnp.dot(x_ref[...], w_ref[...], preferred_element_type=jnp.float32)


@functools.lru_cache(maxsize=None)
def _proj_call(n, din):
    return pl.pallas_call(
        _proj_body,
        out_shape=jax.ShapeDtypeStruct((n, D), jnp.float32),
    )


def _node_mid_body(s_ref, cnt_ref, x_ref, awm_ref, awx_ref, ab_ref,
                   wi_ref, wj_ref, mwx_ref, xo_ref, ti_ref, tj_ref, xm_ref):
    ssum = s_ref[0] + s_ref[1]
    cnt = cnt_ref[0, :, 0:1] + cnt_ref[1, :, 0:1]
    mean = ssum / jnp.maximum(cnt, 1.0)
    h = jnp.dot(mean, awm_ref[...], preferred_element_type=jnp.float32)
    h = h + jnp.dot(x_ref[...], awx_ref[...], preferred_element_type=jnp.float32)
    h = jnp.maximum(h + ab_ref[...], 0.0)
    nrm = jnp.sqrt(jnp.sum(h * h, axis=1, keepdims=True))
    xn = h / jnp.maximum(nrm, 1e-12)
    xo_ref[...] = xn
    ti_ref[...] = jnp.dot(xn, wi_ref[...], preferred_element_type=jnp.float32)
    tj_ref[...] = jnp.dot(xn, wj_ref[...], preferred_element_type=jnp.float32)
    xm_ref[...] = jnp.dot(xn, mwx_ref[...], preferred_element_type=jnp.float32)


@functools.lru_cache(maxsize=None)
def _node_mid_call(n, din):
    return pl.pallas_call(
        _node_mid_body,
        out_shape=[
            jax.ShapeDtypeStruct((n, D), jnp.float32),
            jax.ShapeDtypeStruct((n, 16), jnp.float32),
            jax.ShapeDtypeStruct((n, 16), jnp.float32),
            jax.ShapeDtypeStruct((n, D), jnp.float32),
        ],
    )


def _node_final_body(s_ref, cnt_ref, x_ref, awm_ref, awx_ref, ab_ref,
                     pw0_ref, pb0_ref, pw1_ref, pb1_ref, out_ref):
    ssum = s_ref[0] + s_ref[1]
    cnt = cnt_ref[0, :, 0:1] + cnt_ref[1, :, 0:1]
    mean = ssum / jnp.maximum(cnt, 1.0)
    h = jnp.dot(mean, awm_ref[...], preferred_element_type=jnp.float32)
    h = h + jnp.dot(x_ref[...], awx_ref[...], preferred_element_type=jnp.float32)
    h = jnp.maximum(h + ab_ref[...], 0.0)
    nrm = jnp.sqrt(jnp.sum(h * h, axis=1, keepdims=True))
    xn = h / jnp.maximum(nrm, 1e-12)
    o = jnp.maximum(jnp.dot(xn, pw0_ref[...], preferred_element_type=jnp.float32) + pb0_ref[...], 0.0)
    out_ref[...] = jnp.dot(o, pw1_ref[...], preferred_element_type=jnp.float32) + pb1_ref[...]


@functools.lru_cache(maxsize=None)
def _node_final_call(n):
    return pl.pallas_call(
        _node_final_body,
        out_shape=jax.ShapeDtypeStruct((n, D), jnp.float32),
    )


def kernel(x, edge_attr, edge_index,
           msg_w0, msg_b0, agg_w0, agg_b0,
           msg_w1, msg_b1, agg_w1, agg_b1,
           msg_w2, msg_b2, agg_w2, agg_b2,
           eu_w0, eu_b0, eu_w1, eu_b1, eu_w2, eu_b2,
           post_w0, post_b0, post_w1, post_b1):
    n, din = x.shape
    e = edge_attr.shape[0]
    src = edge_index[0].astype(jnp.int32)
    dst = edge_index[1].astype(jnp.int32)
    z64 = jnp.zeros((n, D), jnp.float32)
    z16 = jnp.zeros((n, 16), jnp.float32)

    r2 = lambda b: b.reshape(1, -1)
    eye8 = jnp.eye(8, dtype=jnp.float32)

    def pack_m(w, b):
        w8 = jnp.stack([jnp.kron(eye8, w[:, 16 * j:16 * j + 16]) for j in range(4)])
        b8 = jnp.stack([jnp.tile(b[16 * j:16 * j + 16], 8) for j in range(4)])
        return w8, b8.reshape(4, 1, 128)

    def pack_e(w, b):
        return jnp.kron(eye8, w), jnp.tile(b, 8).reshape(1, 128)

    ea8 = edge_attr.reshape(e // 8, 128)
    wm80, bm80 = pack_m(msg_w0[din:], msg_b0)
    we80, be80 = pack_e(eu_w0[2 * D:], eu_b0)
    eam0, eae0 = _edge0_call(e)(ea8, wm80, bm80, we80, be80)
    xm0 = _proj_call(n, din)(x, msg_w0[:din])

    part0, cntp = _conv_sc(n, e, True)(src, dst, xm0, eam0, z64, z16)
    x1, t1i, t1j, xm1 = _node_mid_call(n, din)(
        part0, cntp, x, agg_w0[:D], agg_w0[D:], r2(agg_b0),
        eu_w0[:D], eu_w0[D:2 * D], msg_w1[:D])

    wm81, bm81 = pack_m(msg_w1[D:], msg_b1)
    we81, be81 = pack_e(eu_w1[2 * D:], eu_b1)
    (pre1,) = _eupd_sc(n, e)(src, dst, t1i, t1j)
    eam1, eae1 = _edge_mid_call(e, True)(pre1, eae0, wm81, bm81, we81, be81)

    (part1,) = _conv_sc(n, e, False)(src, dst, xm1, eam1, z64, z16)
    x2, t2i, t2j, xm2 = _node_mid_call(n, D)(
        part1, cntp, x1, agg_w1[:D], agg_w1[D:], r2(agg_b1),
        eu_w1[:D], eu_w1[D:2 * D], msg_w2[:D])

    wm82, bm82 = pack_m(msg_w2[D:], msg_b2)
    (pre2,) = _eupd_sc(n, e)(src, dst, t2i, t2j)
    (eam2,) = _edge_mid_call(e, False)(pre2, eae1, wm82, bm82)

    (part2,) = _conv_sc(n, e, False)(src, dst, xm2, eam2, z64, z16)
    out = _node_final_call(n)(
        part2, cntp, x2, agg_w2[:D], agg_w2[D:], r2(agg_b2),
        post_w0, r2(post_b0), post_w1, r2(post_b1))
    return out


# speedup vs baseline: 3.9216x; 1.4689x over previous
import functools

import jax
import jax.numpy as jnp
from jax import lax
from jax.experimental import pallas as pl
from jax.experimental.pallas import tpu as pltpu
from jax.experimental.pallas import tpu_sc as plsc

NC = 2
NS = 16
NW = NC * NS
CH = 128
D = 64


RC = 16


def _row_sweep(n, s, do):
    nrch = n // RC
    rr = nrch // NS
    rrem = nrch - rr * NS

    @pl.loop(0, rr)
    def _(t):
        do((t * NS + s) * RC)

    if rrem:
        @pl.when(s < rrem)
        def _():
            do((rr * NS + s) * RC)


@functools.lru_cache(maxsize=None)
def _conv_sc(n, e):
    assert e % CH == 0 and n % RC == 0
    nchunks = e // CH
    rounds = nchunks // NW
    rem = nchunks - rounds * NW
    assert rounds % 2 == 0 and rounds >= 4

    mesh = plsc.VectorSubcoreMesh(core_axis_name="c", subcore_axis_name="s",
                                  num_cores=NC, num_subcores=NS)
    out_type = [jax.ShapeDtypeStruct((NC, n, D), jnp.float32)]
    scratch = [
        pltpu.VMEM_SHARED((n, D), jnp.float32),
        pltpu.VMEM_SHARED((n, D), jnp.float32),
    ]
    for _ in range(2):
        scratch += [
            pltpu.VMEM((CH,), jnp.int32),
            pltpu.VMEM((CH,), jnp.int32),
            pltpu.VMEM((CH,), jnp.int32),
            pltpu.VMEM((CH, D), jnp.float32),
            pltpu.VMEM((CH, D), jnp.float32),
            pltpu.VMEM((4, CH // 8, 128), jnp.float32),
            pltpu.SemaphoreType.DMA,
            pltpu.SemaphoreType.DMA,
            pltpu.SemaphoreType.DMA,
            pltpu.SemaphoreType.DMA,
        ]

    def body(src_h, dst_h, tab_h, eam_h, z64_h, *rest):
        out_h = rest[0]
        tab_s, acc_s = rest[1], rest[2]
        bufs = [rest[3:13], rest[13:23]]
        c = lax.axis_index("c")
        s = lax.axis_index("s")
        wid = c * NS + s

        def stage(rb):
            sl = pl.ds(rb, RC)
            pltpu.sync_copy(tab_h.at[sl], tab_s.at[sl])
            pltpu.sync_copy(z64_h.at[sl], acc_s.at[sl])

        _row_sweep(n, s, stage)
        plsc.subcore_barrier()

        def base_of(t):
            return jnp.minimum(t * NW + wid, nchunks - 1) * CH

        def issue_idx(t, b):
            sidx, didx = bufs[b][0], bufs[b][1]
            isem = bufs[b][6]
            bs = base_of(t)
            pltpu.async_copy(src_h.at[pl.ds(bs, CH)], sidx, isem)
            pltpu.async_copy(dst_h.at[pl.ds(bs, CH)], didx, isem)

        def wait_idx(b):
            sidx, didx = bufs[b][0], bufs[b][1]
            isem = bufs[b][6]
            pltpu.make_async_copy(src_h.at[pl.ds(0, CH)], sidx, isem).wait()
            pltpu.make_async_copy(dst_h.at[pl.ds(0, CH)], didx, isem).wait()

        def issue_fetch(t, b):
            sidx, g, ebuf = bufs[b][0], bufs[b][3], bufs[b][5]
            gsem, esem = bufs[b][7], bufs[b][8]
            bs = base_of(t)
            pltpu.async_copy(tab_s.at[sidx], g, gsem)
            pltpu.async_copy(eam_h.at[:, pl.ds(bs // 8, CH // 8), :], ebuf, esem)

        def wait_fetch(b):
            sidx, g, ebuf = bufs[b][0], bufs[b][3], bufs[b][5]
            gsem, esem = bufs[b][7], bufs[b][8]
            pltpu.make_async_copy(tab_s.at[sidx], g, gsem).wait()
            pltpu.make_async_copy(
                eam_h.at[:, pl.ds(0, CH // 8), :], ebuf, esem).wait()

        def copy_didx(b):
            didx, sdidx = bufs[b][1], bufs[b][2]

            @pl.loop(0, CH // 16)
            def _(i):
                sdidx[pl.ds(i * 16, 16)] = didx[pl.ds(i * 16, 16)]

        def compute(b):
            g, m, ebuf = bufs[b][3], bufs[b][4], bufs[b][5]

            @pl.loop(0, CH // 8)
            def _(o):
                for u in range(8):
                    k = 8 * o + u
                    for j in range(D // 16):
                        m[k, pl.ds(j * 16, 16)] = jnp.maximum(
                            g[k, pl.ds(j * 16, 16)]
                            + ebuf[j, o, pl.ds(u * 16, 16)], 0.0)

        def issue_scatter(b):
            sdidx, m, ssem = bufs[b][2], bufs[b][4], bufs[b][9]
            pltpu.async_copy(m, acc_s.at[sdidx], ssem, add=True)

        def wait_scatter(b):
            sdidx, m, ssem = bufs[b][2], bufs[b][4], bufs[b][9]
            pltpu.make_async_copy(m, acc_s.at[sdidx], ssem).wait()

        def step(t, b, first):
            nb = 1 - b
            wait_fetch(b)
            wait_idx(nb)
            issue_fetch(t + 1, nb)
            if not first:
                wait_scatter(b)
            copy_didx(b)
            issue_idx(t + 2, b)
            compute(b)
            issue_scatter(b)

        issue_idx(0, 0)
        issue_idx(1, 1)
        wait_idx(0)
        issue_fetch(0, 0)
        step(0, 0, True)
        step(1, 1, True)

        @pl.loop(1, rounds // 2)
        def _(p):
            step(2 * p, 0, False)
            step(2 * p + 1, 1, False)

        wait_fetch(0)
        wait_idx(1)
        wait_scatter(0)
        wait_scatter(1)

        if rem:
            @pl.when(wid < rem)
            def _():
                bs = (rounds * NW + wid) * CH
                sidx, didx, g, m, ebuf = (bufs[0][0], bufs[0][1], bufs[0][3],
                                          bufs[0][4], bufs[0][5])
                sem = bufs[0][7]
                pltpu.sync_copy(src_h.at[pl.ds(bs, CH)], sidx)
                pltpu.sync_copy(dst_h.at[pl.ds(bs, CH)], didx)
                pltpu.async_copy(tab_s.at[sidx], g, sem).wait()
                pltpu.sync_copy(eam_h.at[:, pl.ds(bs // 8, CH // 8), :], ebuf)
                compute(0)
                pltpu.sync_copy(m, acc_s.at[didx], add=True)

        plsc.subcore_barrier()

        def writeback(rb):
            sl = pl.ds(rb, RC)
            pltpu.sync_copy(acc_s.at[sl], out_h.at[c, sl])

        _row_sweep(n, s, writeback)

    return pl.kernel(body, out_type=out_type, mesh=mesh, scratch_types=scratch,
                     compiler_params=pltpu.CompilerParams(use_tc_tiling_on_sc=False))


@functools.lru_cache(maxsize=None)
def _cnt_sc(n, e):
    assert e % CH == 0 and n % RC == 0
    nchunks = e // CH
    rounds = nchunks // NW
    rem = nchunks - rounds * NW
    assert rounds % 2 == 0 and rounds >= 4

    mesh = plsc.VectorSubcoreMesh(core_axis_name="c", subcore_axis_name="s",
                                  num_cores=NC, num_subcores=NS)
    out_type = [jax.ShapeDtypeStruct((NC, n, 16), jnp.float32)]
    scratch = [
        pltpu.VMEM_SHARED((n, 16), jnp.float32),
        pltpu.VMEM((CH, 16), jnp.float32),
    ]
    for _ in range(2):
        scratch += [
            pltpu.VMEM((CH,), jnp.int32),
            pltpu.VMEM((CH,), jnp.int32),
            pltpu.SemaphoreType.DMA,
            pltpu.SemaphoreType.DMA,
        ]

    def body(dst_h, z16_h, out_h, cnt_s, ones, *rest):
        bufs = [rest[0:4], rest[4:8]]
        c = lax.axis_index("c")
        s = lax.axis_index("s")
        wid = c * NS + s

        def stage(rb):
            sl = pl.ds(rb, RC)
            pltpu.sync_copy(z16_h.at[sl], cnt_s.at[sl])

        _row_sweep(n, s, stage)

        @pl.loop(0, CH)
        def _(i):
            ones[i, :] = jnp.full((16,), 1.0, jnp.float32)

        plsc.subcore_barrier()

        def base_of(t):
            return jnp.minimum(t * NW + wid, nchunks - 1) * CH

        def issue_idx(t, b):
            pltpu.async_copy(dst_h.at[pl.ds(base_of(t), CH)], bufs[b][0],
                             bufs[b][2])

        def wait_idx(b):
            pltpu.make_async_copy(dst_h.at[pl.ds(0, CH)], bufs[b][0],
                                  bufs[b][2]).wait()

        def step(t, b, first):
            didx, sdidx, _, ssem = bufs[b]
            wait_idx(b)
            if not first:
                pltpu.make_async_copy(ones, cnt_s.at[sdidx], ssem).wait()

            @pl.loop(0, CH // 16)
            def _(i):
                sdidx[pl.ds(i * 16, 16)] = didx[pl.ds(i * 16, 16)]

            issue_idx(t + 2, b)
            pltpu.async_copy(ones, cnt_s.at[sdidx], ssem, add=True)

        issue_idx(0, 0)
        issue_idx(1, 1)
        step(0, 0, True)
        step(1, 1, True)

        @pl.loop(1, rounds // 2)
        def _(p):
            step(2 * p, 0, False)
            step(2 * p + 1, 1, False)

        wait_idx(0)
        wait_idx(1)
        pltpu.make_async_copy(ones, cnt_s.at[bufs[0][1]], bufs[0][3]).wait()
        pltpu.make_async_copy(ones, cnt_s.at[bufs[1][1]], bufs[1][3]).wait()

        if rem:
            @pl.when(wid < rem)
            def _():
                bs = (rounds * NW + wid) * CH
                pltpu.sync_copy(dst_h.at[pl.ds(bs, CH)], bufs[0][0])
                pltpu.sync_copy(ones, cnt_s.at[bufs[0][0]], add=True)

        plsc.subcore_barrier()

        def writeback(rb):
            sl = pl.ds(rb, RC)
            pltpu.sync_copy(cnt_s.at[sl], out_h.at[c, sl])

        _row_sweep(n, s, writeback)

    return pl.kernel(body, out_type=out_type, mesh=mesh, scratch_types=scratch,
                     compiler_params=pltpu.CompilerParams(use_tc_tiling_on_sc=False))


@functools.lru_cache(maxsize=None)
def _eupd_sc(n, e):
    assert e % CH == 0 and n % RC == 0
    nchunks = e // CH
    rounds = nchunks // NW
    rem = nchunks - rounds * NW

    mesh = plsc.VectorSubcoreMesh(core_axis_name="c", subcore_axis_name="s",
                                  num_cores=NC, num_subcores=NS)
    assert rounds % 2 == 0 and rounds >= 4
    out_type = [jax.ShapeDtypeStruct((e // 8, 128), jnp.float32)]
    scratch = [
        pltpu.VMEM_SHARED((n, 16), jnp.float32),
        pltpu.VMEM_SHARED((n, 16), jnp.float32),
    ]
    for _ in range(2):
        scratch += [
            pltpu.VMEM((CH,), jnp.int32),
            pltpu.VMEM((CH,), jnp.int32),
            pltpu.VMEM((CH, 16), jnp.float32),
            pltpu.VMEM((CH, 16), jnp.float32),
            pltpu.VMEM((CH // 8, 128), jnp.float32),
            pltpu.SemaphoreType.DMA,
            pltpu.SemaphoreType.DMA,
            pltpu.SemaphoreType.DMA,
            pltpu.SemaphoreType.DMA,
        ]

    def body(src_h, dst_h, ti_h, tj_h, out_h, ti_s, tj_s, *rest):
        bufs = [rest[0:9], rest[9:18]]
        c = lax.axis_index("c")
        s = lax.axis_index("s")
        wid = c * NS + s

        def stage(rb):
            sl = pl.ds(rb, RC)
            pltpu.sync_copy(ti_h.at[sl], ti_s.at[sl])
            pltpu.sync_copy(tj_h.at[sl], tj_s.at[sl])

        _row_sweep(n, s, stage)
        plsc.subcore_barrier()

        def base_of(t):
            return jnp.minimum(t * NW + wid, nchunks - 1) * CH

        def issue_idx(t, b):
            sidx, didx, isem = bufs[b][0], bufs[b][1], bufs[b][5]
            bs = base_of(t)
            pltpu.async_copy(src_h.at[pl.ds(bs, CH)], sidx, isem)
            pltpu.async_copy(dst_h.at[pl.ds(bs, CH)], didx, isem)

        def wait_idx(b):
            sidx, didx, isem = bufs[b][0], bufs[b][1], bufs[b][5]
            pltpu.make_async_copy(src_h.at[pl.ds(0, CH)], sidx, isem).wait()
            pltpu.make_async_copy(dst_h.at[pl.ds(0, CH)], didx, isem).wait()

        def issue_fetch(b):
            sidx, didx, gi, gj = bufs[b][0], bufs[b][1], bufs[b][2], bufs[b][3]
            pltpu.async_copy(ti_s.at[sidx], gi, bufs[b][6])
            pltpu.async_copy(tj_s.at[didx], gj, bufs[b][7])

        def wait_fetch(b):
            sidx, didx, gi, gj = bufs[b][0], bufs[b][1], bufs[b][2], bufs[b][3]
            pltpu.make_async_copy(ti_s.at[sidx], gi, bufs[b][6]).wait()
            pltpu.make_async_copy(tj_s.at[didx], gj, bufs[b][7]).wait()

        def compute(b):
            gi, gj, ob = bufs[b][2], bufs[b][3], bufs[b][4]

            @pl.loop(0, CH // 8)
            def _(o):
                for u in range(8):
                    k = 8 * o + u
                    ob[o, pl.ds(u * 16, 16)] = gi[k, :] + gj[k, :]

        def issue_out(t, b):
            ob, osem = bufs[b][4], bufs[b][8]
            bs = base_of(t)
            pltpu.async_copy(ob, out_h.at[pl.ds(bs // 8, CH // 8)], osem)

        def wait_out(b):
            ob, osem = bufs[b][4], bufs[b][8]
            pltpu.make_async_copy(
                ob, out_h.at[pl.ds(0, CH // 8)], osem).wait()

        def step(t, b, first):
            nb = 1 - b
            wait_fetch(b)
            wait_idx(nb)
            issue_fetch(nb)
            if not first:
                wait_out(b)
            issue_idx(t + 2, b)
            compute(b)
            issue_out(t, b)

        issue_idx(0, 0)
        issue_idx(1, 1)
        wait_idx(0)
        issue_fetch(0)
        step(0, 0, True)
        step(1, 1, True)

        @pl.loop(1, rounds // 2)
        def _(p):
            step(2 * p, 0, False)
            step(2 * p + 1, 1, False)

        wait_fetch(0)
        wait_idx(1)
        wait_out(0)
        wait_out(1)

        if rem:
            @pl.when(wid < rem)
            def _():
                bs = (rounds * NW + wid) * CH
                sidx, didx = bufs[0][0], bufs[0][1]
                pltpu.sync_copy(src_h.at[pl.ds(bs, CH)], sidx)
                pltpu.sync_copy(dst_h.at[pl.ds(bs, CH)], didx)
                issue_fetch(0)
                wait_fetch(0)
                compute(0)
                pltpu.sync_copy(bufs[0][4], out_h.at[pl.ds(bs // 8, CH // 8)])

    return pl.kernel(body, out_type=out_type, mesh=mesh, scratch_types=scratch,
                     compiler_params=pltpu.CompilerParams(use_tc_tiling_on_sc=False))


BE = 6400


BP = BE // 8


def _edge0_body(ea_ref, wm8_ref, bm8_ref, we8_ref, be8_ref, eam_ref, eae_ref):
    a8 = ea_ref[...]
    for j in range(4):
        eam_ref[j] = jnp.dot(a8, wm8_ref[j], preferred_element_type=jnp.float32) + bm8_ref[j]
    eae_ref[...] = jnp.dot(a8, we8_ref[...], preferred_element_type=jnp.float32) + be8_ref[...]


def _edge_mid_body2(pre_ref, eae_ref, wm8_ref, bm8_ref, we8_ref, be8_ref,
                    eam_ref, eaen_ref):
    a8 = jnp.maximum(pre_ref[...] + eae_ref[...], 0.0)
    for j in range(4):
        eam_ref[j] = jnp.dot(a8, wm8_ref[j], preferred_element_type=jnp.float32) + bm8_ref[j]
    eaen_ref[...] = jnp.dot(a8, we8_ref[...], preferred_element_type=jnp.float32) + be8_ref[...]


def _edge_mid_body1(pre_ref, eae_ref, wm8_ref, bm8_ref, eam_ref):
    a8 = jnp.maximum(pre_ref[...] + eae_ref[...], 0.0)
    for j in range(4):
        eam_ref[j] = jnp.dot(a8, wm8_ref[j], preferred_element_type=jnp.float32) + bm8_ref[j]


_P8_IN = lambda: pl.BlockSpec((BP, 128), lambda i: (i, 0))
_WM8 = lambda: pl.BlockSpec((4, 128, 128), lambda i: (0, 0, 0))
_BM8 = lambda: pl.BlockSpec((4, 1, 128), lambda i: (0, 0, 0))
_WE8 = lambda: pl.BlockSpec((128, 128), lambda i: (0, 0))
_BE8 = lambda: pl.BlockSpec((1, 128), lambda i: (0, 0))


def _eam_out(e):
    return (pl.BlockSpec((4, BP, 128), lambda i: (0, i, 0)),
            jax.ShapeDtypeStruct((4, e // 8, 128), jnp.float32))


@functools.lru_cache(maxsize=None)
def _edge0_call(e):
    assert e % BE == 0
    eam_spec, eam_shape = _eam_out(e)
    return pl.pallas_call(
        _edge0_body,
        grid=(e // BE,),
        in_specs=[_P8_IN(), _WM8(), _BM8(), _WE8(), _BE8()],
        out_specs=[eam_spec, _P8_IN()],
        out_shape=[eam_shape, jax.ShapeDtypeStruct((e // 8, 128), jnp.float32)],
    )


@functools.lru_cache(maxsize=None)
def _edge_mid_call(e, with_next):
    assert e % BE == 0
    eam_spec, eam_shape = _eam_out(e)
    in_specs = [_P8_IN(), _P8_IN(), _WM8(), _BM8()]
    out_specs = [eam_spec]
    out_shape = [eam_shape]
    if with_next:
        in_specs += [_WE8(), _BE8()]
        out_specs.append(_P8_IN())
        out_shape.append(jax.ShapeDtypeStruct((e // 8, 128), jnp.float32))
    return pl.pallas_call(
        _edge_mid_body2 if with_next else _edge_mid_body1,
        grid=(e // BE,),
        in_specs=in_specs,
        out_specs=out_specs,
        out_shape=out_shape,
    )


def _proj_body(x_ref, w_ref, o_ref):
    o_ref[...] = jnp.dot(x_ref[...], w_ref[...], preferred_element_type=jnp.float32)


@functools.lru_cache(maxsize=None)
def _proj_call(n, din):
    return pl.pallas_call(
        _proj_body,
        out_shape=jax.ShapeDtypeStruct((n, D), jnp.float32),
    )


def _node_mid_body(s_ref, cnt_ref, x_ref, awm_ref, awx_ref, ab_ref,
                   wi_ref, wj_ref, mwx_ref, xo_ref, ti_ref, tj_ref, xm_ref):
    ssum = s_ref[0] + s_ref[1]
    cnt = cnt_ref[0, :, 0:1] + cnt_ref[1, :, 0:1]
    mean = ssum / jnp.maximum(cnt, 1.0)
    h = jnp.dot(mean, awm_ref[...], preferred_element_type=jnp.float32)
    h = h + jnp.dot(x_ref[...], awx_ref[...], preferred_element_type=jnp.float32)
    h = jnp.maximum(h + ab_ref[...], 0.0)
    nrm = jnp.sqrt(jnp.sum(h * h, axis=1, keepdims=True))
    xn = h / jnp.maximum(nrm, 1e-12)
    xo_ref[...] = xn
    ti_ref[...] = jnp.dot(xn, wi_ref[...], preferred_element_type=jnp.float32)
    tj_ref[...] = jnp.dot(xn, wj_ref[...], preferred_element_type=jnp.float32)
    xm_ref[...] = jnp.dot(xn, mwx_ref[...], preferred_element_type=jnp.float32)


@functools.lru_cache(maxsize=None)
def _node_mid_call(n, din):
    return pl.pallas_call(
        _node_mid_body,
        out_shape=[
            jax.ShapeDtypeStruct((n, D), jnp.float32),
            jax.ShapeDtypeStruct((n, 16), jnp.float32),
            jax.ShapeDtypeStruct((n, 16), jnp.float32),
            jax.ShapeDtypeStruct((n, D), jnp.float32),
        ],
    )


def _node_final_body(s_ref, cnt_ref, x_ref, awm_ref, awx_ref, ab_ref,
                     pw0_ref, pb0_ref, pw1_ref, pb1_ref, out_ref):
    ssum = s_ref[0] + s_ref[1]
    cnt = cnt_ref[0, :, 0:1] + cnt_ref[1, :, 0:1]
    mean = ssum / jnp.maximum(cnt, 1.0)
    h = jnp.dot(mean, awm_ref[...], preferred_element_type=jnp.float32)
    h = h + jnp.dot(x_ref[...], awx_ref[...], preferred_element_type=jnp.float32)
    h = jnp.maximum(h + ab_ref[...], 0.0)
    nrm = jnp.sqrt(jnp.sum(h * h, axis=1, keepdims=True))
    xn = h / jnp.maximum(nrm, 1e-12)
    o = jnp.maximum(jnp.dot(xn, pw0_ref[...], preferred_element_type=jnp.float32) + pb0_ref[...], 0.0)
    out_ref[...] = jnp.dot(o, pw1_ref[...], preferred_element_type=jnp.float32) + pb1_ref[...]


@functools.lru_cache(maxsize=None)
def _node_final_call(n):
    return pl.pallas_call(
        _node_final_body,
        out_shape=jax.ShapeDtypeStruct((n, D), jnp.float32),
    )


def kernel(x, edge_attr, edge_index,
           msg_w0, msg_b0, agg_w0, agg_b0,
           msg_w1, msg_b1, agg_w1, agg_b1,
           msg_w2, msg_b2, agg_w2, agg_b2,
           eu_w0, eu_b0, eu_w1, eu_b1, eu_w2, eu_b2,
           post_w0, post_b0, post_w1, post_b1):
    n, din = x.shape
    e = edge_attr.shape[0]
    src = edge_index[0].astype(jnp.int32)
    dst = edge_index[1].astype(jnp.int32)
    z64 = jnp.zeros((n, D), jnp.float32)
    z16 = jnp.zeros((n, 16), jnp.float32)

    r2 = lambda b: b.reshape(1, -1)
    eye8 = jnp.eye(8, dtype=jnp.float32)

    def pack_m(w, b):
        w8 = jnp.stack([jnp.kron(eye8, w[:, 16 * j:16 * j + 16]) for j in range(4)])
        b8 = jnp.stack([jnp.tile(b[16 * j:16 * j + 16], 8) for j in range(4)])
        return w8, b8.reshape(4, 1, 128)

    def pack_e(w, b):
        return jnp.kron(eye8, w), jnp.tile(b, 8).reshape(1, 128)

    ea8 = edge_attr.reshape(e // 8, 128)
    wm80, bm80 = pack_m(msg_w0[din:], msg_b0)
    we80, be80 = pack_e(eu_w0[2 * D:], eu_b0)
    eam0, eae0 = _edge0_call(e)(ea8, wm80, bm80, we80, be80)
    xm0 = _proj_call(n, din)(x, msg_w0[:din])

    (cntp,) = _cnt_sc(n, e)(dst, z16)
    (part0,) = _conv_sc(n, e)(src, dst, xm0, eam0, z64)
    x1, t1i, t1j, xm1 = _node_mid_call(n, din)(
        part0, cntp, x, agg_w0[:D], agg_w0[D:], r2(agg_b0),
        eu_w0[:D], eu_w0[D:2 * D], msg_w1[:D])

    wm81, bm81 = pack_m(msg_w1[D:], msg_b1)
    we81, be81 = pack_e(eu_w1[2 * D:], eu_b1)
    (pre1,) = _eupd_sc(n, e)(src, dst, t1i, t1j)
    eam1, eae1 = _edge_mid_call(e, True)(pre1, eae0, wm81, bm81, we81, be81)

    (part1,) = _conv_sc(n, e)(src, dst, xm1, eam1, z64)
    x2, t2i, t2j, xm2 = _node_mid_call(n, D)(
        part1, cntp, x1, agg_w1[:D], agg_w1[D:], r2(agg_b1),
        eu_w1[:D], eu_w1[D:2 * D], msg_w2[:D])

    wm82, bm82 = pack_m(msg_w2[D:], msg_b2)
    (pre2,) = _eupd_sc(n, e)(src, dst, t2i, t2j)
    (eam2,) = _edge_mid_call(e, False)(pre2, eae1, wm82, bm82)

    (part2,) = _conv_sc(n, e)(src, dst, xm2, eam2, z64)
    out = _node_final_call(n)(
        part2, cntp, x2, agg_w2[:D], agg_w2[D:], r2(agg_b2),
        post_w0, r2(post_b0), post_w1, r2(post_b1))
    return out
